# baseline (speedup 1.0000x reference)
import jax
import jax.numpy as jnp
from jax import lax
from jax.experimental import pallas as pl
from jax.experimental.pallas import tpu as pltpu
from jax.experimental.pallas import tpu_sc as plsc

_EMB = 128
_W = 64
_NW = 32
_BT = 3200
_NPAD = _NW * _BT
_SB = 128
_NSB = _BT // _SB
_NG = _SB // 16
_UB = 32
_NU = _SB // _UB

_OFF = [0, 119, 124, 136, 148, 158, 164, 170, 172]

_RB = 476
_RC = 1076
_ROWS = 1508

_A78 = 0
_A12 = 4
_A56 = 64
_AROWS = 100


def _sc_body(x_hbm, stk_hbm, out_hbm, stg_v, aux_v, ptbl_v, xsb0, xsb1,
             idx_v, outbuf0, outbuf1, xsem, osem):
    pltpu.sync_copy(stk_hbm, stg_v)

    def bf(ref, woff):
        return plsc.bitcast(ref[pl.ds(woff, 16)], jnp.bfloat16)

    def radd(dst_ref, dst_row, a_ref, a_row, b_ref, b_row):
        for cp in range(4):
            o = 16 * cp
            s = bf(a_ref, a_row * _W + o) + bf(b_ref, b_row * _W + o)
            dst_ref[pl.ds(dst_row * _W + o, 16)] = plsc.bitcast(s, jnp.int32)

    for k in range(2):
        for l in range(2):
            radd(aux_v, _A78 + k * 2 + l,
                 stg_v, _OFF[7] + k, stg_v, _OFF[8] + l)

    @plsc.parallel_loop(0, 119, 1, unroll=2)
    def build_a(r0):
        for kl in range(4):
            radd(ptbl_v, r0 * 4 + kl, stg_v, _OFF[0] + r0, aux_v, _A78 + kl)

    def b12(i1, _):
        @plsc.parallel_loop(0, 12, 1, unroll=2)
        def bj(j2):
            radd(aux_v, _A12 + i1 * 12 + j2,
                 stg_v, _OFF[1] + i1, stg_v, _OFF[2] + j2)
        return 0
    lax.fori_loop(0, 5, b12, 0)

    def bb(ij, _):
        @plsc.parallel_loop(0, 10, 1, unroll=2)
        def bk(k4):
            radd(ptbl_v, _RB + ij * 10 + k4,
                 aux_v, _A12 + ij, stg_v, _OFF[4] + k4)
        return 0
    lax.fori_loop(0, 60, bb, 0)

    def b56(i5, _):
        @plsc.parallel_loop(0, 6, 1, unroll=2)
        def bj(j6):
            radd(aux_v, _A56 + i5 * 6 + j6,
                 stg_v, _OFF[5] + i5, stg_v, _OFF[6] + j6)
        return 0
    lax.fori_loop(0, 6, b56, 0)

    def bc(r3, _):
        @plsc.parallel_loop(0, 36, 1, unroll=2)
        def bj(ij56):
            radd(ptbl_v, _RC + r3 * 36 + ij56,
                 stg_v, _OFF[3] + r3, aux_v, _A56 + ij56)
        return 0
    lax.fori_loop(0, 12, bc, 0)

    wid = lax.axis_index("s") * 2 + lax.axis_index("c")
    base = wid * _BT
    iota = lax.broadcasted_iota(jnp.int32, (16,), 0)
    zeros16 = jnp.zeros((16,), jnp.int32)

    pltpu.async_copy(x_hbm.at[:, pl.ds(base, _SB)], xsb0, xsem)

    def do_sb(sb, buf):
        xsb_v = xsb0 if buf == 0 else xsb1
        xsb_n = xsb1 if buf == 0 else xsb0
        off = base + sb * _SB
        pltpu.make_async_copy(
            x_hbm.at[:, pl.ds(off, _SB)], xsb_v, xsem).wait()

        @pl.when(jnp.asarray(sb) + 1 < _NSB)
        def _():
            pltpu.async_copy(
                x_hbm.at[:, pl.ds(off + _SB, _SB)], xsb_n, xsem)

        for g in range(_NG):
            sl = pl.ds(g * 16, 16)
            xv = [xsb_v[i, sl] for i in range(9)]
            idx_v[pl.ds(0 * _SB + g * 16, 16)] = (
                (xv[0] * 2 + xv[7]) * 2 + xv[8]) * _W
            idx_v[pl.ds(1 * _SB + g * 16, 16)] = (
                _RB + (xv[1] * 12 + xv[2]) * 10 + xv[4]) * _W
            idx_v[pl.ds(2 * _SB + g * 16, 16)] = (
                _RC + (xv[3] * 6 + xv[5]) * 6 + xv[6]) * _W

        for q in range(_NU):
            outbuf_v = outbuf0 if q % 2 == 0 else outbuf1
            uoff = off + q * _UB

            @pl.when(jnp.asarray(sb) * _NU + q >= 2)
            def _():
                pltpu.make_async_copy(
                    outbuf_v,
                    out_hbm.at[pl.ds(uoff - 2 * _UB, _UB)], osem).wait()

            @plsc.parallel_loop(0, _UB, 1, unroll=4)
            def atom_loop(j):
                jf = zeros16 + (q * _UB + j)
                rA = plsc.load_gather(idx_v, [jf])
                rB = plsc.load_gather(idx_v, [jf + _SB])
                rC = plsc.load_gather(idx_v, [jf + 2 * _SB])
                for cp in range(4):
                    colp = iota + 16 * cp
                    aA = plsc.bitcast(
                        plsc.load_gather(ptbl_v, [rA + colp]), jnp.bfloat16)
                    aB = plsc.bitcast(
                        plsc.load_gather(ptbl_v, [rB + colp]), jnp.bfloat16)
                    aC = plsc.bitcast(
                        plsc.load_gather(ptbl_v, [rC + colp]), jnp.bfloat16)
                    s = (aA + aB) + aC
                    lo, hi = plsc.unpack(s, format=plsc.PackFormat.INTERLEAVED)
                    outbuf_v[j, pl.ds(32 * cp, 16)] = lo
                    outbuf_v[j, pl.ds(32 * cp + 16, 16)] = hi
            pltpu.async_copy(outbuf_v, out_hbm.at[pl.ds(uoff, _UB)], osem)

    def pair_loop(i2, _):
        do_sb(i2 * 2, 0)
        do_sb(i2 * 2 + 1, 1)
        return 0
    lax.fori_loop(0, _NSB // 2, pair_loop, 0)
    for sb in range(2 * (_NSB // 2), _NSB):
        do_sb(sb, 0)
    for b in (outbuf0, outbuf1):
        pltpu.make_async_copy(b, out_hbm.at[pl.ds(base, _UB)], osem).wait()


@jax.jit
def kernel(x_0, table_0, table_1, table_2, table_3, table_4, table_5,
           table_6, table_7, table_8):
    n = x_0.shape[0]
    xT = jnp.pad(x_0, ((0, _NPAD - n), (0, 0))).T
    stk = jnp.concatenate(
        [table_0, table_1, table_2, table_3, table_4, table_5, table_6,
         table_7, table_8], axis=0)
    pairs = stk.astype(jnp.bfloat16).reshape(174, 4, 2, 16).transpose(
        0, 1, 3, 2)
    stkp = jax.lax.bitcast_convert_type(pairs, jnp.int32).reshape(-1)
    mesh = plsc.VectorSubcoreMesh(core_axis_name="c", subcore_axis_name="s")
    fn = pl.kernel(
        _sc_body,
        out_type=jax.ShapeDtypeStruct((_NPAD, _EMB), jnp.float32),
        mesh=mesh,
        compiler_params=pltpu.CompilerParams(needs_layout_passes=False),
        scratch_types=[
            pltpu.VMEM((174 * _W,), jnp.int32),
            pltpu.VMEM((_AROWS * _W,), jnp.int32),
            pltpu.VMEM((_ROWS * _W,), jnp.int32),
            pltpu.VMEM((9, _SB), jnp.int32),
            pltpu.VMEM((9, _SB), jnp.int32),
            pltpu.VMEM((3 * _SB,), jnp.int32),
            pltpu.VMEM((_UB, _EMB), jnp.float32),
            pltpu.VMEM((_UB, _EMB), jnp.float32),
            pltpu.SemaphoreType.DMA,
            pltpu.SemaphoreType.DMA,
        ],
    )
    out = fn(xT, stkp)
    return out[:n]
